# baseline (speedup 1.0000x reference)
import functools

import jax
import jax.numpy as jnp
from jax import lax
from jax.experimental import pallas as pl
from jax.experimental.pallas import tpu as pltpu
from jax.experimental.pallas import tpu_sc as plsc

_NUM_CLASSES = 10
_K = 8
_N = 1024
_D = 16
_M = 100000
_BLK = 128
_CHUNK = 2048
_CHUNK_B = 4096
_MPAD = 102400
_NCHUNK = _MPAD // _CHUNK
_NCHUNK_B = _MPAD // _CHUNK_B
_BPC = _CHUNK_B // _BLK
_NBLK = _MPAD // _BLK
_NCAND = _K * _BLK
_NROWS = _N * _K
_BIGF = float(3.0e38)
_BIGI = 2**31 - 1
_PADV = float(1.0e4)


def _dist_a_kernel(x_ref, yt_ref, x2_ref, y2_ref, d_ref):
    mm = jnp.dot(x_ref[...], yt_ref[...], preferred_element_type=jnp.float32)
    d_ref[...] = x2_ref[...] + y2_ref[...] - 2.0 * mm


def _dist_b_kernel(y_ref, xt_ref, y2_ref, x2_ref, b_ref):
    mm = jnp.dot(y_ref[...], xt_ref[...], preferred_element_type=jnp.float32)
    d2 = x2_ref[...] + y2_ref[...] - 2.0 * mm
    b_ref[...] = jnp.min(d2.reshape(_BPC, _BLK, _N), axis=1)


def _select_blocks_kernel(b_ref, out_ref):
    b = jnp.sqrt(jnp.maximum(b_ref[...], 0.0))
    ids = jax.lax.broadcasted_iota(jnp.int32, b.shape, 0)
    rows = []
    for _ in range(_K):
        m = jnp.min(b, axis=0, keepdims=True)
        sel = jnp.min(jnp.where(b == m, ids, _BIGI), axis=0, keepdims=True)
        rows.append(sel)
        b = jnp.where(ids == sel, _BIGF, b)
    out_ref[...] = jnp.concatenate(rows, axis=0)


def _make_sc_gather():
    info = plsc.get_sparse_core_info()
    nw = info.num_cores * info.num_subcores
    gpw = _NROWS // nw
    mesh = plsc.VectorSubcoreMesh(core_axis_name="c", subcore_axis_name="s")

    @functools.partial(
        pl.kernel,
        mesh=mesh,
        out_type=[
            jax.ShapeDtypeStruct((_NROWS, _BLK), jnp.float32),
            jax.ShapeDtypeStruct((_NROWS, _BLK), jnp.int32),
        ],
        scratch_types=[
            pltpu.VMEM((gpw,), jnp.int32),
            pltpu.VMEM((gpw,), jnp.int32),
            pltpu.VMEM((gpw, _BLK), jnp.float32),
            pltpu.VMEM((gpw, _BLK), jnp.int32),
            pltpu.SemaphoreType.DMA,
            pltpu.SemaphoreType.DMA,
        ],
    )
    def sc_gather(d2d_hbm, lab2d_hbm, rq_hbm, blk_hbm, outd_hbm, outl_hbm,
                  idx_v, idx2_v, rows_v, lrows_v, sem1, sem2):
        wid = lax.axis_index("s") * info.num_cores + lax.axis_index("c")
        base = wid * gpw
        pltpu.sync_copy(rq_hbm.at[pl.ds(base, gpw)], idx_v)
        pltpu.sync_copy(blk_hbm.at[pl.ds(base, gpw)], idx2_v)
        cp1 = pltpu.async_copy(d2d_hbm.at[idx_v], rows_v, sem1)
        cp2 = pltpu.async_copy(lab2d_hbm.at[idx2_v], lrows_v, sem2)
        cp1.wait()
        cp2.wait()
        pltpu.sync_copy(rows_v, outd_hbm.at[pl.ds(base, gpw)])
        pltpu.sync_copy(lrows_v, outl_hbm.at[pl.ds(base, gpw)])

    return sc_gather


def _topk_vote_kernel(cd_ref, cg_ref, cl_ref, w_ref):
    d = jnp.sqrt(jnp.maximum(cd_ref[...], 0.0))
    g = cg_ref[...]
    lab = cl_ref[...]
    counts = [jnp.zeros((_N, 1), jnp.int32) for _ in range(_NUM_CLASSES)]
    for _ in range(_K):
        m = jnp.min(d, axis=1, keepdims=True)
        gsel = jnp.min(jnp.where(d == m, g, _BIGI), axis=1, keepdims=True)
        hit = g == gsel
        lsel = jnp.min(jnp.where(hit, lab, _BIGI), axis=1, keepdims=True)
        d = jnp.where(hit, _BIGF, d)
        for c in range(_NUM_CLASSES):
            counts[c] = counts[c] + (lsel == c).astype(jnp.int32)
    winner = jnp.zeros((_N, 1), jnp.int32)
    count = jnp.full((_N, 1), -1, jnp.int32)
    for labv in range(_NUM_CLASSES):
        vc = counts[labv]
        who = vc >= count
        winner = jnp.where(who, labv, winner)
        count = jnp.where(who, vc, count)
    w_ref[...] = winner


def kernel(x, train_pts, train_label):
    f32 = jnp.float32
    ypad = jnp.concatenate(
        [train_pts, jnp.full((_MPAD - _M, _D), _PADV, f32)], axis=0)
    labpad = jnp.concatenate(
        [train_label.astype(jnp.int32),
         jnp.zeros((_MPAD - _M,), jnp.int32)], axis=0)
    x2c = jnp.sum(x * x, axis=1, keepdims=True)
    y2r = jnp.sum(ypad * ypad, axis=1)[None, :]
    yt = ypad.T
    xt = x.T
    x2r = x2c.T
    y2c = y2r.T

    dist = pl.pallas_call(
        _dist_a_kernel,
        grid=(_NCHUNK,),
        in_specs=[
            pl.BlockSpec((_N, _D), lambda i: (0, 0)),
            pl.BlockSpec((_D, _CHUNK), lambda i: (0, i)),
            pl.BlockSpec((_N, 1), lambda i: (0, 0)),
            pl.BlockSpec((1, _CHUNK), lambda i: (0, i)),
        ],
        out_specs=pl.BlockSpec((_N, _CHUNK), lambda i: (0, i)),
        out_shape=jax.ShapeDtypeStruct((_N, _MPAD), f32),
    )(x, yt, x2c, y2r)

    bmin = pl.pallas_call(
        _dist_b_kernel,
        grid=(_NCHUNK_B,),
        in_specs=[
            pl.BlockSpec((_CHUNK_B, _D), lambda i: (i, 0)),
            pl.BlockSpec((_D, _N), lambda i: (0, 0)),
            pl.BlockSpec((_CHUNK_B, 1), lambda i: (i, 0)),
            pl.BlockSpec((1, _N), lambda i: (0, 0)),
        ],
        out_specs=pl.BlockSpec((_BPC, _N), lambda i: (i, 0)),
        out_shape=jax.ShapeDtypeStruct((_NBLK, _N), f32),
    )(ypad, xt, y2c, x2r)

    blk8 = pl.pallas_call(
        _select_blocks_kernel,
        out_shape=jax.ShapeDtypeStruct((_K, _N), jnp.int32),
    )(bmin)

    blk8t = blk8.T
    rq = (jnp.arange(_N, dtype=jnp.int32)[:, None] * _NBLK
          + blk8t).reshape(_NROWS)
    blkflat = blk8t.reshape(_NROWS)

    cand_d2, cand_l2 = _make_sc_gather()(
        dist.reshape(_N * _NBLK, _BLK),
        labpad.reshape(_NBLK, _BLK),
        rq,
        blkflat,
    )
    cand_d = cand_d2.reshape(_N, _NCAND)
    cand_l = cand_l2.reshape(_N, _NCAND)
    pid = (blk8t[:, :, None] * _BLK
           + jnp.arange(_BLK, dtype=jnp.int32)[None, None, :]).reshape(
               _N, _NCAND)

    winner = pl.pallas_call(
        _topk_vote_kernel,
        out_shape=jax.ShapeDtypeStruct((_N, 1), jnp.int32),
    )(cand_d, pid, cand_l)

    return winner[:, 0].astype(train_label.dtype)


# speedup vs baseline: 1.0664x; 1.0664x over previous
import functools

import jax
import jax.numpy as jnp
from jax import lax
from jax.experimental import pallas as pl
from jax.experimental.pallas import tpu as pltpu
from jax.experimental.pallas import tpu_sc as plsc

_NUM_CLASSES = 10
_K = 8
_N = 1024
_D = 16
_M = 100000
_BLK = 128
_CHUNK = 2048
_MPAD = 100352
_NCHUNK = _MPAD // _CHUNK
_NCHUNK_B = _NCHUNK
_CHUNK_B = _CHUNK
_BPC = _CHUNK_B // _BLK
_NBLK = _MPAD // _BLK
_NCAND = _K * _BLK
_NROWS = _N * _K
_BIGF = float(3.0e38)
_BIGI = 2**31 - 1
_PADV = float(1.0e4)


def _dist_a_kernel(x_ref, yt_ref, x2_ref, y2_ref, d_ref):
    mm = jnp.dot(x_ref[...], yt_ref[...], preferred_element_type=jnp.float32)
    d_ref[...] = x2_ref[...] + y2_ref[...] - 2.0 * mm


def _dist_b_kernel(y_ref, xt_ref, y2_ref, x2_ref, b_ref):
    mm = jnp.dot(y_ref[...], xt_ref[...], preferred_element_type=jnp.float32)
    d2 = x2_ref[...] + y2_ref[...] - 2.0 * mm
    b_ref[...] = jnp.min(d2.reshape(_BPC, _BLK, _N), axis=1)


def _select_blocks_kernel(b_ref, out_ref):
    b = jnp.sqrt(jnp.maximum(b_ref[...], 0.0))
    ids = jax.lax.broadcasted_iota(jnp.int32, b.shape, 0)
    rows = []
    for _ in range(_K):
        m = jnp.min(b, axis=0, keepdims=True)
        sel = jnp.min(jnp.where(b == m, ids, _BIGI), axis=0, keepdims=True)
        rows.append(sel)
        b = jnp.where(ids == sel, _BIGF, b)
    out_ref[...] = jnp.concatenate(rows, axis=0)


def _make_sc_gather():
    info = plsc.get_sparse_core_info()
    nw = info.num_cores * info.num_subcores
    gpw = _NROWS // nw
    mesh = plsc.VectorSubcoreMesh(core_axis_name="c", subcore_axis_name="s")

    @functools.partial(
        pl.kernel,
        mesh=mesh,
        out_type=[
            jax.ShapeDtypeStruct((_NROWS, _BLK), jnp.float32),
            jax.ShapeDtypeStruct((_NROWS, _BLK), jnp.int32),
        ],
        scratch_types=[
            pltpu.VMEM((gpw,), jnp.int32),
            pltpu.VMEM((gpw,), jnp.int32),
            pltpu.VMEM((gpw, _BLK), jnp.float32),
            pltpu.VMEM((gpw, _BLK), jnp.int32),
            pltpu.SemaphoreType.DMA,
            pltpu.SemaphoreType.DMA,
        ],
    )
    def sc_gather(d2d_hbm, lab2d_hbm, rq_hbm, blk_hbm, outd_hbm, outl_hbm,
                  idx_v, idx2_v, rows_v, lrows_v, sem1, sem2):
        wid = lax.axis_index("s") * info.num_cores + lax.axis_index("c")
        base = wid * gpw
        pltpu.sync_copy(rq_hbm.at[pl.ds(base, gpw)], idx_v)
        pltpu.sync_copy(blk_hbm.at[pl.ds(base, gpw)], idx2_v)
        cp1 = pltpu.async_copy(d2d_hbm.at[idx_v], rows_v, sem1)
        cp2 = pltpu.async_copy(lab2d_hbm.at[idx2_v], lrows_v, sem2)
        cp1.wait()
        cp2.wait()
        pltpu.sync_copy(rows_v, outd_hbm.at[pl.ds(base, gpw)])
        pltpu.sync_copy(lrows_v, outl_hbm.at[pl.ds(base, gpw)])

    return sc_gather


def _topk_vote_kernel(cd_ref, cg_ref, cl_ref, w_ref):
    d = jnp.sqrt(jnp.maximum(cd_ref[...], 0.0))
    g = cg_ref[...]
    lab = cl_ref[...]
    counts = [jnp.zeros((_N, 1), jnp.int32) for _ in range(_NUM_CLASSES)]
    for _ in range(_K):
        m = jnp.min(d, axis=1, keepdims=True)
        gsel = jnp.min(jnp.where(d == m, g, _BIGI), axis=1, keepdims=True)
        hit = g == gsel
        lsel = jnp.min(jnp.where(hit, lab, _BIGI), axis=1, keepdims=True)
        d = jnp.where(hit, _BIGF, d)
        for c in range(_NUM_CLASSES):
            counts[c] = counts[c] + (lsel == c).astype(jnp.int32)
    winner = jnp.zeros((_N, 1), jnp.int32)
    count = jnp.full((_N, 1), -1, jnp.int32)
    for labv in range(_NUM_CLASSES):
        vc = counts[labv]
        who = vc >= count
        winner = jnp.where(who, labv, winner)
        count = jnp.where(who, vc, count)
    w_ref[...] = winner


def kernel(x, train_pts, train_label):
    f32 = jnp.float32
    ypad = jnp.concatenate(
        [train_pts, jnp.full((_MPAD - _M, _D), _PADV, f32)], axis=0)
    labpad = jnp.concatenate(
        [train_label.astype(jnp.int32),
         jnp.zeros((_MPAD - _M,), jnp.int32)], axis=0)
    x2c = jnp.sum(x * x, axis=1, keepdims=True)
    y2r = jnp.sum(ypad * ypad, axis=1)[None, :]
    yt = ypad.T
    xt = x.T
    x2r = x2c.T
    y2c = y2r.T

    dist = pl.pallas_call(
        _dist_a_kernel,
        grid=(_NCHUNK,),
        in_specs=[
            pl.BlockSpec((_N, _D), lambda i: (0, 0)),
            pl.BlockSpec((_D, _CHUNK), lambda i: (0, i)),
            pl.BlockSpec((_N, 1), lambda i: (0, 0)),
            pl.BlockSpec((1, _CHUNK), lambda i: (0, i)),
        ],
        out_specs=pl.BlockSpec((_N, _CHUNK), lambda i: (0, i)),
        out_shape=jax.ShapeDtypeStruct((_N, _MPAD), f32),
    )(x, yt, x2c, y2r)

    bmin = pl.pallas_call(
        _dist_b_kernel,
        grid=(_NCHUNK_B,),
        in_specs=[
            pl.BlockSpec((_CHUNK_B, _D), lambda i: (i, 0)),
            pl.BlockSpec((_D, _N), lambda i: (0, 0)),
            pl.BlockSpec((_CHUNK_B, 1), lambda i: (i, 0)),
            pl.BlockSpec((1, _N), lambda i: (0, 0)),
        ],
        out_specs=pl.BlockSpec((_BPC, _N), lambda i: (i, 0)),
        out_shape=jax.ShapeDtypeStruct((_NBLK, _N), f32),
    )(ypad, xt, y2c, x2r)

    blk8 = pl.pallas_call(
        _select_blocks_kernel,
        out_shape=jax.ShapeDtypeStruct((_K, _N), jnp.int32),
    )(bmin)

    blk8t = blk8.T
    rq = (jnp.arange(_N, dtype=jnp.int32)[:, None] * _NBLK
          + blk8t).reshape(_NROWS)
    blkflat = blk8t.reshape(_NROWS)

    cand_d2, cand_l2 = _make_sc_gather()(
        dist.reshape(_N * _NBLK, _BLK),
        labpad.reshape(_NBLK, _BLK),
        rq,
        blkflat,
    )
    cand_d = cand_d2.reshape(_N, _NCAND)
    cand_l = cand_l2.reshape(_N, _NCAND)
    pid = (blk8t[:, :, None] * _BLK
           + jnp.arange(_BLK, dtype=jnp.int32)[None, None, :]).reshape(
               _N, _NCAND)

    winner = pl.pallas_call(
        _topk_vote_kernel,
        out_shape=jax.ShapeDtypeStruct((_N, 1), jnp.int32),
    )(cand_d, pid, cand_l)

    return winner[:, 0].astype(train_label.dtype)
